# baseline (speedup 1.0000x reference)
import functools

import jax
import jax.numpy as jnp
from jax.experimental import pallas as pl
from jax.experimental.pallas import tpu as pltpu

N = 10000
E = 320000
T = 3
D_IN = 128
HID = 16
HEADS = 8
D = HID * HEADS
SEM_H = 128
OUT = 64


def _final_mm_kernel(h_ref, w_ref, b_ref, o_ref):
    o_ref[...] = h_ref[...] @ w_ref[...] + b_ref[...]


def _final_mm(h2, Wf, bf):
    n = h2.shape[0]
    return pl.pallas_call(
        _final_mm_kernel,
        out_shape=jax.ShapeDtypeStruct((n, OUT), jnp.float32),
        grid=(10,),
        in_specs=[
            pl.BlockSpec((n // 10, D), lambda i: (i, 0)),
            pl.BlockSpec((D, OUT), lambda i: (0, 0)),
            pl.BlockSpec((1, OUT), lambda i: (0, 0)),
        ],
        out_specs=pl.BlockSpec((n // 10, OUT), lambda i: (i, 0)),
    )(h2, Wf, bf.reshape(1, OUT))


def _gatv2(h, src, dst, Wsrc, bsrc, Wdst, bdst, attn):
    n = h.shape[0]
    fs = (h @ Wsrc + bsrc).reshape(n, HEADS, HID)
    fd = (h @ Wdst + bdst).reshape(n, HEADS, HID)
    e = fs[src] + fd[dst]
    e = jnp.where(e > 0, e, 0.2 * e)
    logits = jnp.einsum('ehf,hf->eh', e, attn)
    m = jax.ops.segment_max(logits, dst, num_segments=n)
    m = jnp.where(jnp.isfinite(m), m, 0.0)
    ex = jnp.exp(logits - m[dst])
    s = jax.ops.segment_sum(ex, dst, num_segments=n)
    alpha = ex / (s[dst] + 1e-9)
    out = jax.ops.segment_sum(fs[src] * alpha[:, :, None], dst, num_segments=n)
    out = jnp.where(out > 0, out, jnp.expm1(out))
    return out.reshape(n, HEADS * HID)


def _sem_att(z, W1, b1, W2):
    w = (jnp.tanh(z @ W1 + b1) @ W2).mean(0)
    beta = jax.nn.softmax(w, axis=0)
    return (beta[None, :, :] * z).sum(1)


def kernel(h, edge_index_0, edge_index_1, edge_index_2, node_nums, Wsrc0, bsrc0, Wdst0, bdst0, attn0, Sw1_0, Sb1_0, Sw2_0, Wsrc1, bsrc1, Wdst1, bdst1, attn1, Sw1_1, Sb1_1, Sw2_1, Wf, bf):
    eis = [edge_index_0, edge_index_1, edge_index_2]
    z = jnp.stack([_gatv2(h, eis[t][0], eis[t][1], Wsrc0[t], bsrc0[t], Wdst0[t], bdst0[t], attn0[t]) for t in range(T)], axis=1)
    h1 = _sem_att(z, Sw1_0, Sb1_0, Sw2_0)
    z = jnp.stack([_gatv2(h1, eis[t][0], eis[t][1], Wsrc1[t], bsrc1[t], Wdst1[t], bdst1[t], attn1[t]) for t in range(T)], axis=1)
    h2 = _sem_att(z, Sw1_1, Sb1_1, Sw2_1)
    return _final_mm(h2, Wf, bf)


# speedup vs baseline: 20.7684x; 20.7684x over previous
import functools

import jax
import jax.numpy as jnp
from jax import lax
from jax.experimental import pallas as pl
from jax.experimental.pallas import tpu as pltpu
from jax.experimental.pallas import tpu_sc as plsc

N = 10000
E = 320000
T = 3
D_IN = 128
HID = 16
HEADS = 8
D = HID * HEADS
SEM_H = 128
OUT = 64

NW = 32
EPW = E // NW
C = 80
NCHUNK = EPW // C
NPAD = 10240
RPT = NPAD // 16
ZR = 128
SCOLS = 144
NB = 10
RB = N // NB

_HIGH = lax.Precision.HIGHEST


def _dot(a, b):
    return jnp.dot(a, b, preferred_element_type=jnp.float32, precision=_HIGH)


def _edge_body(fs_hbm, fd_hbm, src_hbm, dst_hbm, attn_hbm, out_hbm,
               src_c, dst_c, rows_s, rows_d, wbuf,
               attn_v, lbuf, acc, sem_s, sem_d):
    c = lax.axis_index("c")
    s = lax.axis_index("s")
    w = c * 16 + s
    ebase = w * EPW

    pltpu.sync_copy(attn_hbm, attn_v)

    zvec = jnp.zeros((16,), jnp.float32)

    def _zero_w(i, carry):
        for k in range(SCOLS // 16):
            wbuf[i, pl.ds(16 * k, 16)] = zvec
        return carry

    lax.fori_loop(0, C, _zero_w, 0)

    for r in range(RPT // C):
        pltpu.sync_copy(wbuf, acc.at[pl.ds(s * RPT + r * C, C)])
    plsc.subcore_barrier()

    def _chunk(ci, carry):
        base = ebase + ci * C
        pltpu.sync_copy(src_hbm.at[pl.ds(base, C)], src_c)
        pltpu.sync_copy(dst_hbm.at[pl.ds(base, C)], dst_c)
        cp1 = pltpu.async_copy(fs_hbm.at[src_c], rows_s, sem_s)
        cp2 = pltpu.async_copy(fd_hbm.at[dst_c], rows_d, sem_d)
        cp1.wait()
        cp2.wait()
        lane15 = lax.iota(jnp.int32, 16) == 15
        for g in range(C // 16):
            def _edge_logits(e, carry2):
                eg = g * 16 + e
                ev = jnp.full((16,), e, jnp.int32)
                for h in range(HEADS):
                    x = rows_s[eg, pl.ds(16 * h, 16)] + rows_d[eg, pl.ds(16 * h, 16)]
                    t = jnp.maximum(x, 0.2 * x)
                    cs = plsc.cumsum(t * attn_v[h])
                    plsc.store_scatter(
                        lbuf, [jnp.full((16,), h, jnp.int32), ev], cs, mask=lane15)
                return carry2

            lax.fori_loop(0, 16, _edge_logits, 0)

            eidx = lax.iota(jnp.int32, 16) + g * 16
            wvs = []
            for h in range(HEADS):
                wv = jnp.exp(lbuf[h])
                wvs.append(wv)
                plsc.store_scatter(
                    wbuf, [eidx, jnp.full((16,), 128 + h, jnp.int32)], wv)

            def _edge_rows(e, carry2):
                eg = g * 16 + e
                ev = jnp.full((16,), e, jnp.int32)
                for h in range(HEADS):
                    bc = jnp.take_along_axis(wvs[h], ev, axis=0)
                    wbuf[eg, pl.ds(16 * h, 16)] = rows_s[eg, pl.ds(16 * h, 16)] * bc
                return carry2

            lax.fori_loop(0, 16, _edge_rows, 0)
        pltpu.sync_copy(wbuf, acc.at[dst_c], add=True)
        return carry

    lax.fori_loop(0, NCHUNK, _chunk, 0)
    plsc.subcore_barrier()
    pltpu.sync_copy(acc.at[pl.ds(s * RPT, RPT)],
                    out_hbm.at[c, pl.ds(s * RPT, RPT)])


_edge_call = pl.kernel(
    _edge_body,
    out_type=jax.ShapeDtypeStruct((2, NPAD, SCOLS), jnp.float32),
    mesh=plsc.VectorSubcoreMesh(core_axis_name="c", subcore_axis_name="s"),
    compiler_params=pltpu.CompilerParams(needs_layout_passes=False,
                                         use_tc_tiling_on_sc=False),
    scratch_types=[
        pltpu.VMEM((C,), jnp.int32),
        pltpu.VMEM((C,), jnp.int32),
        pltpu.VMEM((C, D), jnp.float32),
        pltpu.VMEM((C, D), jnp.float32),
        pltpu.VMEM((C, SCOLS), jnp.float32),
        pltpu.VMEM((HEADS, HID), jnp.float32),
        pltpu.VMEM((HEADS, 16), jnp.float32),
        pltpu.VMEM_SHARED((NPAD, SCOLS), jnp.float32),
        pltpu.SemaphoreType.DMA,
        pltpu.SemaphoreType.DMA,
    ],
)


def _proj_body(x_ref, ws_ref, bs_ref, wd_ref, bd_ref, fs_ref, fd_ref):
    x = x_ref[...]
    fs_ref[0] = _dot(x, ws_ref[0]) + bs_ref[0]
    fd_ref[0] = _dot(x, wd_ref[0]) + bd_ref[0]


def _proj(x, Ws, bs, Wd, bd):
    bs = bs.reshape(T, 1, D)
    bd = bd.reshape(T, 1, D)
    return pl.pallas_call(
        _proj_body,
        out_shape=(jax.ShapeDtypeStruct((T, N, D), jnp.float32),
                   jax.ShapeDtypeStruct((T, N, D), jnp.float32)),
        grid=(T, NB),
        in_specs=[
            pl.BlockSpec((RB, D_IN), lambda t, i: (i, 0)),
            pl.BlockSpec((1, D_IN, D), lambda t, i: (t, 0, 0)),
            pl.BlockSpec((1, 1, D), lambda t, i: (t, 0, 0)),
            pl.BlockSpec((1, D_IN, D), lambda t, i: (t, 0, 0)),
            pl.BlockSpec((1, 1, D), lambda t, i: (t, 0, 0)),
        ],
        out_specs=(pl.BlockSpec((1, RB, D), lambda t, i: (t, i, 0)),
                   pl.BlockSpec((1, RB, D), lambda t, i: (t, i, 0))),
    )(x, Ws, bs, Wd, bd)


def _proj2_body(z_ref, beta_ref, ws_ref, bs_ref, wd_ref, bd_ref, fs_ref, fd_ref):
    x = (beta_ref[0, 0] * z_ref[:, 0, :]
         + beta_ref[0, 1] * z_ref[:, 1, :]
         + beta_ref[0, 2] * z_ref[:, 2, :])
    fs_ref[0] = _dot(x, ws_ref[0]) + bs_ref[0]
    fd_ref[0] = _dot(x, wd_ref[0]) + bd_ref[0]


def _proj2(z, beta, Ws, bs, Wd, bd):
    bs = bs.reshape(T, 1, D)
    bd = bd.reshape(T, 1, D)
    return pl.pallas_call(
        _proj2_body,
        out_shape=(jax.ShapeDtypeStruct((T, N, D), jnp.float32),
                   jax.ShapeDtypeStruct((T, N, D), jnp.float32)),
        grid=(T, NB),
        in_specs=[
            pl.BlockSpec((RB, T, D), lambda t, i: (i, 0, 0)),
            pl.BlockSpec((1, T), lambda t, i: (0, 0)),
            pl.BlockSpec((1, D, D), lambda t, i: (t, 0, 0)),
            pl.BlockSpec((1, 1, D), lambda t, i: (t, 0, 0)),
            pl.BlockSpec((1, D, D), lambda t, i: (t, 0, 0)),
            pl.BlockSpec((1, 1, D), lambda t, i: (t, 0, 0)),
        ],
        out_specs=(pl.BlockSpec((1, RB, D), lambda t, i: (t, i, 0)),
                   pl.BlockSpec((1, RB, D), lambda t, i: (t, i, 0))),
    )(z, beta, Ws, bs, Wd, bd)


def _combine_body(a0_ref, a1_ref, a2_ref, w1_ref, b1_ref, w2_ref, z_ref, wp_ref):
    rep = (lax.broadcasted_iota(jnp.int32, (HEADS, D), 1) // HID
           == lax.broadcasted_iota(jnp.int32, (HEADS, D), 0)).astype(jnp.float32)
    parts = []
    for t, a in enumerate((a0_ref, a1_ref, a2_ref)):
        acc = a[0] + a[1]
        num = acc[:, :D]
        sv = acc[:, D:D + HEADS]
        den = _dot(sv, rep) + 1e-9
        o = num / den
        zt = jnp.where(o > 0, o, jnp.exp(o) - 1.0)
        z_ref[:, t, :] = zt
        u = jnp.tanh(_dot(zt, w1_ref[...]) + b1_ref[...])
        parts.append(jnp.sum(u * w2_ref[...]))
    wp_ref[0, 0, :] = jnp.stack(parts)


def _combine(a0, a1, a2, Sw1, Sb1, Sw2row):
    return pl.pallas_call(
        _combine_body,
        out_shape=(jax.ShapeDtypeStruct((N, T, D), jnp.float32),
                   jax.ShapeDtypeStruct((NB, 1, T), jnp.float32)),
        grid=(NB,),
        in_specs=[
            pl.BlockSpec((2, RB, SCOLS), lambda i: (0, i, 0)),
            pl.BlockSpec((2, RB, SCOLS), lambda i: (0, i, 0)),
            pl.BlockSpec((2, RB, SCOLS), lambda i: (0, i, 0)),
            pl.BlockSpec((SEM_H, SEM_H), lambda i: (0, 0)),
            pl.BlockSpec((1, SEM_H), lambda i: (0, 0)),
            pl.BlockSpec((1, SEM_H), lambda i: (0, 0)),
        ],
        out_specs=(pl.BlockSpec((RB, T, D), lambda i: (i, 0, 0)),
                   pl.BlockSpec((1, 1, T), lambda i: (i, 0, 0))),
    )(a0, a1, a2, Sw1, Sb1.reshape(1, SEM_H), Sw2row)


def _final_body(z_ref, beta_ref, wf_ref, bf_ref, o_ref):
    x = (beta_ref[0, 0] * z_ref[:, 0, :]
         + beta_ref[0, 1] * z_ref[:, 1, :]
         + beta_ref[0, 2] * z_ref[:, 2, :])
    o_ref[...] = _dot(x, wf_ref[...]) + bf_ref[...]


def _final(z, beta, Wf, bf):
    return pl.pallas_call(
        _final_body,
        out_shape=jax.ShapeDtypeStruct((N, OUT), jnp.float32),
        grid=(NB,),
        in_specs=[
            pl.BlockSpec((RB, T, D), lambda i: (i, 0, 0)),
            pl.BlockSpec((1, T), lambda i: (0, 0)),
            pl.BlockSpec((D, OUT), lambda i: (0, 0)),
            pl.BlockSpec((1, OUT), lambda i: (0, 0)),
        ],
        out_specs=pl.BlockSpec((RB, OUT), lambda i: (i, 0)),
    )(z, beta, Wf, bf.reshape(1, OUT))


def _layer(x_z, eis, proj_fn, proj_args, attn, Sw1, Sb1, Sw2):
    fs_all, fd_all = proj_fn(*proj_args)
    accs = [_edge_call(fs_all[t], fd_all[t], eis[t][0], eis[t][1], attn[t])
            for t in range(T)]
    z, wp = _combine(accs[0], accs[1], accs[2], Sw1, Sb1, Sw2.reshape(1, SEM_H))
    beta = jax.nn.softmax(wp.sum(axis=(0, 1)) / N).reshape(1, T)
    return z, beta


def kernel(h, edge_index_0, edge_index_1, edge_index_2, node_nums, Wsrc0, bsrc0, Wdst0, bdst0, attn0, Sw1_0, Sb1_0, Sw2_0, Wsrc1, bsrc1, Wdst1, bdst1, attn1, Sw1_1, Sb1_1, Sw2_1, Wf, bf):
    eis = [edge_index_0, edge_index_1, edge_index_2]
    z1, beta1 = _layer(h, eis, _proj, (h, Wsrc0, bsrc0, Wdst0, bdst0),
                       attn0, Sw1_0, Sb1_0, Sw2_0)
    z2, beta2 = _layer(None, eis, _proj2, (z1, beta1, Wsrc1, bsrc1, Wdst1, bdst1),
                       attn1, Sw1_1, Sb1_1, Sw2_1)
    return _final(z2, beta2, Wf, bf)
